# baseline (speedup 1.0000x reference)
import functools

import jax
import jax.numpy as jnp
from jax import lax
from jax.experimental import pallas as pl
from jax.experimental.pallas import tpu as pltpu
from jax.experimental.pallas import tpu_sc as plsc

_N = 100000
_E = 3200000
_NW = 32
_RANGE = _N // _NW
_TILE = 2000
_NTILES = _E // _TILE
_BATCH = 256
_PCAP = 2304
_LANES = 16


def _tables_body(u_ref, w_ref, a_ref, p_ref):
    ap = jnp.dot(u_ref[...], w_ref[...], preferred_element_type=jnp.float32)
    a_ref[...] = ap[:, :16]
    p_ref[...] = ap[:, 16:]


def _build_tables(u, wbig):
    n = u.shape[0]
    blk = 1000
    return pl.pallas_call(
        _tables_body,
        grid=(n // blk,),
        in_specs=[
            pl.BlockSpec((blk, 24), lambda i: (i, 0)),
            pl.BlockSpec((24, 32), lambda i: (0, 0)),
        ],
        out_specs=[
            pl.BlockSpec((blk, 16), lambda i: (i, 0)),
            pl.BlockSpec((blk, 16), lambda i: (i, 0)),
        ],
        out_shape=[
            jax.ShapeDtypeStruct((n, 16), jnp.float32),
            jax.ShapeDtypeStruct((n, 16), jnp.float32),
        ],
    )(u, wbig)


def _final_body(e_f, raw_ref, sums_ref, sq_ref, g_ref, b_ref, o_ref):
    s = jnp.sum(sums_ref[...], axis=0, keepdims=True)
    q = jnp.sum(sq_ref[...], axis=0, keepdims=True)
    mean = s / e_f
    var = q / e_f - mean * mean
    rstd = lax.rsqrt(var + 1e-5)
    raw = raw_ref[...]
    aff = (raw - mean) * (rstd * g_ref[...]) + b_ref[...]
    o_ref[...] = jnp.where(raw == -jnp.inf, raw, aff)


def _finalize(raw, sums, sumsqs, gamma, beta, e_f):
    n = raw.shape[0]
    blk = 1000
    return pl.pallas_call(
        functools.partial(_final_body, e_f),
        grid=(n // blk,),
        in_specs=[
            pl.BlockSpec((blk, 16), lambda i: (i, 0)),
            pl.BlockSpec((_NW, 16), lambda i: (0, 0)),
            pl.BlockSpec((_NW, 16), lambda i: (0, 0)),
            pl.BlockSpec((1, 16), lambda i: (0, 0)),
            pl.BlockSpec((1, 16), lambda i: (0, 0)),
        ],
        out_specs=pl.BlockSpec((blk, 16), lambda i: (i, 0)),
        out_shape=jax.ShapeDtypeStruct((n, 16), jnp.float32),
    )(raw, sums, sumsqs, gamma, beta)


def _sc_body(src_hbm, dst_hbm, a_hbm, p_hbm,
             pooled_hbm, part_hbm,
             sbuf0, sbuf1, dbuf0, dbuf1,
             pend_s, pend_d, idx_s, idx_d,
             ba, bp, acc, sums, fill_s,
             sem_s0, sem_s1, sem_d0, sem_d1, sem_a, sem_p):
    cid = lax.axis_index("c")
    sid = lax.axis_index("s")
    wid = sid * 2 + cid
    lo = wid * _RANGE
    iota = lax.iota(jnp.int32, _LANES)

    neg = jnp.full((_LANES,), -jnp.inf, jnp.float32)
    zero = jnp.zeros((_LANES,), jnp.float32)

    def _init(i, c):
        acc[pl.ds(i * _LANES, _LANES)] = neg
        return c
    lax.fori_loop(0, _RANGE, _init, 0)
    sums[pl.ds(0, _LANES)] = zero
    sums[pl.ds(_LANES, _LANES)] = zero
    fill_s[0] = 0

    def _tilno(t):
        tt = t + wid * (_NTILES // _NW)
        return jnp.where(tt >= _NTILES, tt - _NTILES, tt)

    def _scopy(t, buf, sem):
        return pltpu.make_async_copy(
            src_hbm.at[pl.ds(_tilno(t) * _TILE, _TILE)], buf, sem)

    def _dcopy(t, buf, sem):
        return pltpu.make_async_copy(
            dst_hbm.at[pl.ds(_tilno(t) * _TILE, _TILE)], buf, sem)

    _gd = lax.GatherDimensionNumbers(
        offset_dims=(), collapsed_slice_dims=(0,), start_index_map=(0,))

    def _lane_bcast(vec, e_rel):
        idx = jnp.full((_LANES, 1), e_rel, jnp.int32)
        return lax.gather(vec, idx, _gd, (1,),
                          mode=lax.GatherScatterMode.PROMISE_IN_BOUNDS)

    def _edge_update(e_rel, loc16, off):
        sb = _lane_bcast(loc16, e_rel)
        a_row = ba[off + e_rel, :]
        p_row = bp[off + e_rel, :]
        msg = a_row - p_row
        sums[pl.ds(0, _LANES)] = sums[pl.ds(0, _LANES)] + msg
        sums[pl.ds(_LANES, _LANES)] = sums[pl.ds(_LANES, _LANES)] + msg * msg
        ridx = sb * _LANES + iota
        old = plsc.load_gather(acc, [ridx])
        plsc.store_scatter(acc, [ridx], jnp.maximum(old, msg))

    def _process_batch(off, masked):
        fv = fill_s[0]
        for v in range(_BATCH // _LANES):
            sv = pend_s[pl.ds(off + v * _LANES, _LANES)]
            dv = pend_d[pl.ds(off + v * _LANES, _LANES)]
            if masked:
                mlane = (iota + v * _LANES) < fv
                sv = jnp.where(mlane, sv, lo)
                dv = jnp.where(mlane, dv, 0)
            idx_s[pl.ds(v * _LANES, _LANES)] = sv
            idx_d[pl.ds(v * _LANES, _LANES)] = dv
        cpa = pltpu.make_async_copy(a_hbm.at[idx_d], ba, sem_a)
        cpp = pltpu.make_async_copy(p_hbm.at[idx_s], bp, sem_p)
        cpa.start()
        cpp.start()
        cpa.wait()
        cpp.wait()

        def _vblock(v, c):
            loc16 = idx_s[pl.ds(v * _LANES, _LANES)] - lo
            for k in range(_LANES):
                if masked:
                    @pl.when(v * _LANES + k < fv)
                    def _(k=k, v=v, loc16=loc16):
                        _edge_update(k, loc16, v * _LANES)
                else:
                    _edge_update(k, loc16, v * _LANES)
            return c
        lax.fori_loop(0, _BATCH // _LANES, _vblock, 0)

    def _drain():
        nb = fill_s[0] // _BATCH

        def _b(j, c):
            _process_batch(j * _BATCH, False)
            return c
        lax.fori_loop(0, nb, _b, 0)

        @pl.when(nb > 0)
        def _():
            roff = nb * _BATCH
            for k in range(_BATCH // _LANES):
                pend_s[pl.ds(k * _LANES, _LANES)] = \
                    pend_s[pl.ds(roff + k * _LANES, _LANES)]
                pend_d[pl.ds(k * _LANES, _LANES)] = \
                    pend_d[pl.ds(roff + k * _LANES, _LANES)]
            fill_s[0] = fill_s[0] - roff

    def _scan_tile(sbuf, dbuf):
        def _v(v, c):
            sv = sbuf[pl.ds(v * _LANES, _LANES)]
            dv = dbuf[pl.ds(v * _LANES, _LANES)]
            b = ((sv.astype(jnp.float32) + 0.5) *
                 (1.0 / _RANGE)).astype(jnp.int32)
            m = b == wid

            @pl.when(jnp.any(m))
            def _():
                f = fill_s[0]
                plsc.store_compressed(pend_s.at[pl.ds(f, _LANES)], sv, mask=m)
                plsc.store_compressed(pend_d.at[pl.ds(f, _LANES)], dv, mask=m)
                fill_s[0] = f + jnp.sum(m.astype(jnp.int32))
            return c
        lax.fori_loop(0, _TILE // _LANES, _v, 0)

    _scopy(0, sbuf0, sem_s0).start()
    _dcopy(0, dbuf0, sem_d0).start()
    _scopy(1, sbuf1, sem_s1).start()
    _dcopy(1, dbuf1, sem_d1).start()

    def _pair(i, c):
        t0 = i * 2
        for b in range(2):
            t = t0 + b
            sbuf = sbuf0 if b == 0 else sbuf1
            dbuf = dbuf0 if b == 0 else dbuf1
            sem_s = sem_s0 if b == 0 else sem_s1
            sem_d = sem_d0 if b == 0 else sem_d1
            _scopy(t, sbuf, sem_s).wait()
            _dcopy(t, dbuf, sem_d).wait()
            _scan_tile(sbuf, dbuf)

            @pl.when(t + 2 < _NTILES)
            def _(t=t, sbuf=sbuf, dbuf=dbuf, sem_s=sem_s, sem_d=sem_d):
                _scopy(t + 2, sbuf, sem_s).start()
                _dcopy(t + 2, dbuf, sem_d).start()
            _drain()
        return c
    lax.fori_loop(0, _NTILES // 2, _pair, 0)

    @pl.when(fill_s[0] > 0)
    def _():
        _process_batch(0, True)

    pltpu.sync_copy(acc, pooled_hbm.at[pl.ds(lo * _LANES, _RANGE * _LANES)])
    pltpu.sync_copy(sums, part_hbm.at[pl.ds(wid * 2 * _LANES, 2 * _LANES)])


def _sc_edge_pass(src, dst, a_tab, p_tab):
    mesh = plsc.VectorSubcoreMesh(core_axis_name="c", subcore_axis_name="s")
    f = pl.kernel(
        _sc_body,
        mesh=mesh,
        compiler_params=pltpu.CompilerParams(
            needs_layout_passes=False, use_tc_tiling_on_sc=False),
        out_type=[
            jax.ShapeDtypeStruct((_N * _LANES,), jnp.float32),
            jax.ShapeDtypeStruct((_NW * 2 * _LANES,), jnp.float32),
        ],
        scratch_types=[
            pltpu.VMEM((_TILE,), jnp.int32),
            pltpu.VMEM((_TILE,), jnp.int32),
            pltpu.VMEM((_TILE,), jnp.int32),
            pltpu.VMEM((_TILE,), jnp.int32),
            pltpu.VMEM((_PCAP,), jnp.int32),
            pltpu.VMEM((_PCAP,), jnp.int32),
            pltpu.VMEM((_BATCH,), jnp.int32),
            pltpu.VMEM((_BATCH,), jnp.int32),
            pltpu.VMEM((_BATCH, _LANES), jnp.float32),
            pltpu.VMEM((_BATCH, _LANES), jnp.float32),
            pltpu.VMEM((_RANGE * _LANES,), jnp.float32),
            pltpu.VMEM((2 * _LANES,), jnp.float32),
            pltpu.SMEM((8,), jnp.int32),
            pltpu.SemaphoreType.DMA,
            pltpu.SemaphoreType.DMA,
            pltpu.SemaphoreType.DMA,
            pltpu.SemaphoreType.DMA,
            pltpu.SemaphoreType.DMA,
            pltpu.SemaphoreType.DMA,
        ],
    )
    return f(src, dst, a_tab, p_tab)


def kernel(node, features, edges, W, gamma, beta):
    n = node.shape[0]
    e = edges.shape[0]
    src = edges[:, 0]
    dst = edges[:, 1]

    u = jnp.concatenate(
        [features, node, jnp.zeros((n, 5), jnp.float32)], axis=1)
    wbig = jnp.zeros((24, 32), jnp.float32)
    wbig = wbig.at[:19, :16].set(W.T)
    wbig = wbig.at[16:19, 16:].set(W[:, 16:19].T)

    a_tab, p_tab = _build_tables(u, wbig)
    pooled_flat, partials = _sc_edge_pass(src, dst, a_tab, p_tab)
    raw = pooled_flat.reshape(n, 16)
    part = partials.reshape(_NW, 2, _LANES)
    out = _finalize(raw, part[:, 0, :], part[:, 1, :],
                    gamma.reshape(1, 16), beta.reshape(1, 16), float(e))

    present = raw[:, 0] > -jnp.inf

    def _compact(o):
        idxs = jnp.nonzero(present, size=n, fill_value=0)[0]
        cnt = jnp.sum(present.astype(jnp.int32))
        o2 = jnp.take(o, idxs, axis=0)
        return jnp.where((jnp.arange(n) < cnt)[:, None], o2, -jnp.inf)

    return lax.cond(jnp.all(present), lambda o: o, _compact, out)


# speedup vs baseline: 1.1206x; 1.1206x over previous
import functools

import jax
import jax.numpy as jnp
from jax import lax
from jax.experimental import pallas as pl
from jax.experimental.pallas import tpu as pltpu
from jax.experimental.pallas import tpu_sc as plsc

_N = 100000
_E = 3200000
_NW = 32
_RANGE = _N // _NW
_TILE = 2000
_NTILES = _E // _TILE
_BATCH = 256
_PCAP = 2304
_LANES = 16


def _tables_body(u_ref, w_ref, a_ref, p_ref):
    ap = jnp.dot(u_ref[...], w_ref[...], preferred_element_type=jnp.float32)
    a_ref[...] = ap[:, :16]
    p_ref[...] = ap[:, 16:]


def _build_tables(u, wbig):
    n = u.shape[0]
    blk = 1000
    return pl.pallas_call(
        _tables_body,
        grid=(n // blk,),
        in_specs=[
            pl.BlockSpec((blk, 24), lambda i: (i, 0)),
            pl.BlockSpec((24, 32), lambda i: (0, 0)),
        ],
        out_specs=[
            pl.BlockSpec((blk, 16), lambda i: (i, 0)),
            pl.BlockSpec((blk, 16), lambda i: (i, 0)),
        ],
        out_shape=[
            jax.ShapeDtypeStruct((n, 16), jnp.float32),
            jax.ShapeDtypeStruct((n, 16), jnp.float32),
        ],
    )(u, wbig)


def _final_body(e_f, raw_ref, sums_ref, sq_ref, g_ref, b_ref, o_ref):
    s = jnp.sum(sums_ref[...], axis=0, keepdims=True)
    q = jnp.sum(sq_ref[...], axis=0, keepdims=True)
    mean = s / e_f
    var = q / e_f - mean * mean
    rstd = lax.rsqrt(var + 1e-5)
    raw = raw_ref[...]
    aff = (raw - mean) * (rstd * g_ref[...]) + b_ref[...]
    o_ref[...] = jnp.where(raw == -jnp.inf, raw, aff)


def _finalize(raw, sums, sumsqs, gamma, beta, e_f):
    n = raw.shape[0]
    blk = 1000
    return pl.pallas_call(
        functools.partial(_final_body, e_f),
        grid=(n // blk,),
        in_specs=[
            pl.BlockSpec((blk, 16), lambda i: (i, 0)),
            pl.BlockSpec((_NW, 16), lambda i: (0, 0)),
            pl.BlockSpec((_NW, 16), lambda i: (0, 0)),
            pl.BlockSpec((1, 16), lambda i: (0, 0)),
            pl.BlockSpec((1, 16), lambda i: (0, 0)),
        ],
        out_specs=pl.BlockSpec((blk, 16), lambda i: (i, 0)),
        out_shape=jax.ShapeDtypeStruct((n, 16), jnp.float32),
    )(raw, sums, sumsqs, gamma, beta)


def _sc_body(src_hbm, dst_hbm, a_hbm, p_hbm,
             pooled_hbm, part_hbm,
             sbuf0, sbuf1, dbuf0, dbuf1,
             pend_s, pend_d,
             idx_s0, idx_d0, ba0, bp0,
             idx_s1, idx_d1, ba1, bp1,
             acc, sums, fill_s,
             sem_s0, sem_s1, sem_d0, sem_d1,
             sem_a0, sem_p0, sem_a1, sem_p1):
    cid = lax.axis_index("c")
    sid = lax.axis_index("s")
    wid = sid * 2 + cid
    lo = wid * _RANGE
    iota = lax.iota(jnp.int32, _LANES)
    sets = ((idx_s0, idx_d0, ba0, bp0, sem_a0, sem_p0),
            (idx_s1, idx_d1, ba1, bp1, sem_a1, sem_p1))

    neg = jnp.full((_LANES,), -jnp.inf, jnp.float32)
    zero = jnp.zeros((_LANES,), jnp.float32)

    def _init(i, c):
        acc[pl.ds(i * _LANES, _LANES)] = neg
        return c
    lax.fori_loop(0, _RANGE, _init, 0)
    sums[pl.ds(0, _LANES)] = zero
    sums[pl.ds(_LANES, _LANES)] = zero
    fill_s[0] = 0

    def _tilno(t):
        tt = t + wid * (_NTILES // _NW)
        return jnp.where(tt >= _NTILES, tt - _NTILES, tt)

    def _scopy(t, buf, sem):
        return pltpu.make_async_copy(
            src_hbm.at[pl.ds(_tilno(t) * _TILE, _TILE)], buf, sem)

    def _dcopy(t, buf, sem):
        return pltpu.make_async_copy(
            dst_hbm.at[pl.ds(_tilno(t) * _TILE, _TILE)], buf, sem)

    _gd = lax.GatherDimensionNumbers(
        offset_dims=(), collapsed_slice_dims=(0,), start_index_map=(0,))

    def _lane_bcast(vec, e_rel):
        idx = jnp.full((_LANES, 1), e_rel, jnp.int32)
        return lax.gather(vec, idx, _gd, (1,),
                          mode=lax.GatherScatterMode.PROMISE_IN_BOUNDS)

    def _stage_start(off, st):
        idx_s, idx_d, ba, bp, sem_a, sem_p = st
        for v in range(_BATCH // _LANES):
            idx_s[pl.ds(v * _LANES, _LANES)] = \
                pend_s[pl.ds(off + v * _LANES, _LANES)]
            idx_d[pl.ds(v * _LANES, _LANES)] = \
                pend_d[pl.ds(off + v * _LANES, _LANES)]
        pltpu.make_async_copy(a_hbm.at[idx_d], ba, sem_a).start()
        pltpu.make_async_copy(p_hbm.at[idx_s], bp, sem_p).start()

    def _wait(st):
        idx_s, idx_d, ba, bp, sem_a, sem_p = st
        pltpu.make_async_copy(a_hbm.at[idx_d], ba, sem_a).wait()
        pltpu.make_async_copy(p_hbm.at[idx_s], bp, sem_p).wait()

    def _edge_update(e_rel, loc16, off, ba, bp):
        sb = _lane_bcast(loc16, e_rel)
        a_row = ba[off + e_rel, :]
        p_row = bp[off + e_rel, :]
        msg = a_row - p_row
        sums[pl.ds(0, _LANES)] = sums[pl.ds(0, _LANES)] + msg
        sums[pl.ds(_LANES, _LANES)] = sums[pl.ds(_LANES, _LANES)] + msg * msg
        ridx = sb * _LANES + iota
        old = plsc.load_gather(acc, [ridx])
        plsc.store_scatter(acc, [ridx], jnp.maximum(old, msg))

    def _process(st, masked):
        idx_s, idx_d, ba, bp, _, _ = st
        fv = fill_s[0]

        def _vblock(v, c):
            loc16 = idx_s[pl.ds(v * _LANES, _LANES)] - lo
            for k in range(_LANES):
                if masked:
                    @pl.when(v * _LANES + k < fv)
                    def _(k=k, v=v, loc16=loc16):
                        _edge_update(k, loc16, v * _LANES, ba, bp)
                else:
                    _edge_update(k, loc16, v * _LANES, ba, bp)
            return c
        lax.fori_loop(0, _BATCH // _LANES, _vblock, 0)

    def _drain():
        nb = fill_s[0] // _BATCH

        @pl.when(nb > 0)
        def _():
            _stage_start(0, sets[0])

            @pl.when(nb > 1)
            def _():
                _stage_start(_BATCH, sets[1])

            def _pairb(k, c):
                for b2 in range(2):
                    j0 = k * 2 + b2

                    @pl.when(j0 < nb)
                    def _(j0=j0, b2=b2):
                        _wait(sets[b2])
                        _process(sets[b2], False)

                        @pl.when(j0 + 2 < nb)
                        def _(j0=j0, b2=b2):
                            _stage_start((j0 + 2) * _BATCH, sets[b2])
                return c
            lax.fori_loop(0, (nb + 1) // 2, _pairb, 0)

            roff = nb * _BATCH
            for k in range(_BATCH // _LANES):
                pend_s[pl.ds(k * _LANES, _LANES)] = \
                    pend_s[pl.ds(roff + k * _LANES, _LANES)]
                pend_d[pl.ds(k * _LANES, _LANES)] = \
                    pend_d[pl.ds(roff + k * _LANES, _LANES)]
            fill_s[0] = fill_s[0] - roff

    def _scan_tile(sbuf, dbuf):
        def _v(vi, c):
            for u in range(5):
                v_off = (vi * 5 + u) * _LANES
                sv = sbuf[pl.ds(v_off, _LANES)]
                dv = dbuf[pl.ds(v_off, _LANES)]
                b = ((sv.astype(jnp.float32) + 0.5) *
                     (1.0 / _RANGE)).astype(jnp.int32)
                m = b == wid
                pc = plsc.all_reduce_population_count(m)[0]

                @pl.when(pc > 0)
                def _(sv=sv, dv=dv, m=m, pc=pc):
                    f = fill_s[0]
                    plsc.store_compressed(pend_s.at[pl.ds(f, _LANES)],
                                          sv, mask=m)
                    plsc.store_compressed(pend_d.at[pl.ds(f, _LANES)],
                                          dv, mask=m)
                    fill_s[0] = f + pc
            return c
        lax.fori_loop(0, _TILE // _LANES // 5, _v, 0)

    _scopy(0, sbuf0, sem_s0).start()
    _dcopy(0, dbuf0, sem_d0).start()
    _scopy(1, sbuf1, sem_s1).start()
    _dcopy(1, dbuf1, sem_d1).start()

    def _pair(i, c):
        t0 = i * 2
        for b in range(2):
            t = t0 + b
            sbuf = sbuf0 if b == 0 else sbuf1
            dbuf = dbuf0 if b == 0 else dbuf1
            sem_s = sem_s0 if b == 0 else sem_s1
            sem_d = sem_d0 if b == 0 else sem_d1
            _scopy(t, sbuf, sem_s).wait()
            _dcopy(t, dbuf, sem_d).wait()
            _scan_tile(sbuf, dbuf)

            @pl.when(t + 2 < _NTILES)
            def _(t=t, sbuf=sbuf, dbuf=dbuf, sem_s=sem_s, sem_d=sem_d):
                _scopy(t + 2, sbuf, sem_s).start()
                _dcopy(t + 2, dbuf, sem_d).start()
            _drain()
        return c
    lax.fori_loop(0, _NTILES // 2, _pair, 0)

    @pl.when(fill_s[0] > 0)
    def _():
        fv = fill_s[0]
        for v in range(_BATCH // _LANES):
            mlane = (iota + v * _LANES) < fv
            sv = pend_s[pl.ds(v * _LANES, _LANES)]
            dv = pend_d[pl.ds(v * _LANES, _LANES)]
            pend_s[pl.ds(v * _LANES, _LANES)] = jnp.where(mlane, sv, lo)
            pend_d[pl.ds(v * _LANES, _LANES)] = jnp.where(mlane, dv, 0)
        _stage_start(0, sets[0])
        _wait(sets[0])
        _process(sets[0], True)

    pltpu.sync_copy(acc, pooled_hbm.at[pl.ds(lo * _LANES, _RANGE * _LANES)])
    pltpu.sync_copy(sums, part_hbm.at[pl.ds(wid * 2 * _LANES, 2 * _LANES)])


def _sc_edge_pass(src, dst, a_tab, p_tab):
    mesh = plsc.VectorSubcoreMesh(core_axis_name="c", subcore_axis_name="s")
    f = pl.kernel(
        _sc_body,
        mesh=mesh,
        compiler_params=pltpu.CompilerParams(
            needs_layout_passes=False, use_tc_tiling_on_sc=False),
        out_type=[
            jax.ShapeDtypeStruct((_N * _LANES,), jnp.float32),
            jax.ShapeDtypeStruct((_NW * 2 * _LANES,), jnp.float32),
        ],
        scratch_types=[
            pltpu.VMEM((_TILE,), jnp.int32),
            pltpu.VMEM((_TILE,), jnp.int32),
            pltpu.VMEM((_TILE,), jnp.int32),
            pltpu.VMEM((_TILE,), jnp.int32),
            pltpu.VMEM((_PCAP,), jnp.int32),
            pltpu.VMEM((_PCAP,), jnp.int32),
            pltpu.VMEM((_BATCH,), jnp.int32),
            pltpu.VMEM((_BATCH,), jnp.int32),
            pltpu.VMEM((_BATCH, _LANES), jnp.float32),
            pltpu.VMEM((_BATCH, _LANES), jnp.float32),
            pltpu.VMEM((_BATCH,), jnp.int32),
            pltpu.VMEM((_BATCH,), jnp.int32),
            pltpu.VMEM((_BATCH, _LANES), jnp.float32),
            pltpu.VMEM((_BATCH, _LANES), jnp.float32),
            pltpu.VMEM((_RANGE * _LANES,), jnp.float32),
            pltpu.VMEM((2 * _LANES,), jnp.float32),
            pltpu.SMEM((8,), jnp.int32),
            pltpu.SemaphoreType.DMA,
            pltpu.SemaphoreType.DMA,
            pltpu.SemaphoreType.DMA,
            pltpu.SemaphoreType.DMA,
            pltpu.SemaphoreType.DMA,
            pltpu.SemaphoreType.DMA,
            pltpu.SemaphoreType.DMA,
            pltpu.SemaphoreType.DMA,
        ],
    )
    return f(src, dst, a_tab, p_tab)


def kernel(node, features, edges, W, gamma, beta):
    n = node.shape[0]
    e = edges.shape[0]
    src = edges[:, 0]
    dst = edges[:, 1]

    u = jnp.concatenate(
        [features, node, jnp.zeros((n, 5), jnp.float32)], axis=1)
    wbig = jnp.zeros((24, 32), jnp.float32)
    wbig = wbig.at[:19, :16].set(W.T)
    wbig = wbig.at[16:19, 16:].set(W[:, 16:19].T)

    a_tab, p_tab = _build_tables(u, wbig)
    pooled_flat, partials = _sc_edge_pass(src, dst, a_tab, p_tab)
    raw = pooled_flat.reshape(n, 16)
    part = partials.reshape(_NW, 2, _LANES)
    out = _finalize(raw, part[:, 0, :], part[:, 1, :],
                    gamma.reshape(1, 16), beta.reshape(1, 16), float(e))

    present = raw[:, 0] > -jnp.inf

    def _compact(o):
        idxs = jnp.nonzero(present, size=n, fill_value=0)[0]
        cnt = jnp.sum(present.astype(jnp.int32))
        o2 = jnp.take(o, idxs, axis=0)
        return jnp.where((jnp.arange(n) < cnt)[:, None], o2, -jnp.inf)

    return lax.cond(jnp.all(present), lambda o: o, _compact, out)


# speedup vs baseline: 3.4244x; 3.0558x over previous
import functools

import jax
import jax.numpy as jnp
from jax import lax
from jax.experimental import pallas as pl
from jax.experimental.pallas import tpu as pltpu
from jax.experimental.pallas import tpu_sc as plsc

_N = 100000
_E = 3200000
_NW = 32
_RANGE = _N // _NW
_CHUNK = _E // _NW
_TILE = 2000
_NT2 = _CHUNK // _TILE
_BATCH = 256
_CAP = 544
_REGCAP = _CHUNK + 576
_LANES = 16

_SC_PARAMS = pltpu.CompilerParams(
    needs_layout_passes=False, use_tc_tiling_on_sc=False)
_GD = lax.GatherDimensionNumbers(
    offset_dims=(), collapsed_slice_dims=(0,), start_index_map=(0,))


def _vgather(vec, idx):
    return lax.gather(vec, idx.reshape(_LANES, 1), _GD, (1,),
                      mode=lax.GatherScatterMode.PROMISE_IN_BOUNDS)


def _tables_body(u_ref, w_ref, a_ref, p_ref):
    ap = jnp.dot(u_ref[...], w_ref[...], preferred_element_type=jnp.float32)
    a_ref[...] = ap[:, :16]
    p_ref[...] = ap[:, 16:]


def _build_tables(u, wbig):
    n = u.shape[0]
    blk = 1000
    return pl.pallas_call(
        _tables_body,
        grid=(n // blk,),
        in_specs=[
            pl.BlockSpec((blk, 24), lambda i: (i, 0)),
            pl.BlockSpec((24, 32), lambda i: (0, 0)),
        ],
        out_specs=[
            pl.BlockSpec((blk, 16), lambda i: (i, 0)),
            pl.BlockSpec((blk, 16), lambda i: (i, 0)),
        ],
        out_shape=[
            jax.ShapeDtypeStruct((n, 16), jnp.float32),
            jax.ShapeDtypeStruct((n, 16), jnp.float32),
        ],
    )(u, wbig)


def _final_body(e_f, raw_ref, sums_ref, sq_ref, g_ref, b_ref, o_ref):
    s = jnp.sum(sums_ref[...], axis=0, keepdims=True)
    q = jnp.sum(sq_ref[...], axis=0, keepdims=True)
    mean = s / e_f
    var = q / e_f - mean * mean
    rstd = lax.rsqrt(var + 1e-5)
    raw = raw_ref[...]
    aff = (raw - mean) * (rstd * g_ref[...]) + b_ref[...]
    o_ref[...] = jnp.where(raw == -jnp.inf, raw, aff)


def _finalize(raw, sums, sumsqs, gamma, beta, e_f):
    n = raw.shape[0]
    blk = 1000
    return pl.pallas_call(
        functools.partial(_final_body, e_f),
        grid=(n // blk,),
        in_specs=[
            pl.BlockSpec((blk, 16), lambda i: (i, 0)),
            pl.BlockSpec((_NW, 16), lambda i: (0, 0)),
            pl.BlockSpec((_NW, 16), lambda i: (0, 0)),
            pl.BlockSpec((1, 16), lambda i: (0, 0)),
            pl.BlockSpec((1, 16), lambda i: (0, 0)),
        ],
        out_specs=pl.BlockSpec((blk, 16), lambda i: (i, 0)),
        out_shape=jax.ShapeDtypeStruct((n, 16), jnp.float32),
    )(raw, sums, sumsqs, gamma, beta)


def _part_body(src_hbm, dst_hbm,
               bsrc_hbm, bdst_hbm, cnts_hbm,
               sbuf0, sbuf1, dbuf0, dbuf1,
               stg_s, stg_d, cur, dc, cstage,
               sem_s0, sem_s1, sem_d0, sem_d1):
    cid = lax.axis_index("c")
    sid = lax.axis_index("s")
    wid = sid * 2 + cid
    iota = lax.iota(jnp.int32, _LANES)
    bvec0 = iota * _CAP
    bvec1 = (iota + _LANES) * _CAP

    cur[pl.ds(0, _LANES)] = bvec0
    cur[pl.ds(_LANES, _LANES)] = bvec1
    dc[pl.ds(0, _LANES)] = jnp.zeros((_LANES,), jnp.int32)
    dc[pl.ds(_LANES, _LANES)] = jnp.zeros((_LANES,), jnp.int32)

    cbase = wid * _CHUNK

    def _scopy(t, buf, sem):
        return pltpu.make_async_copy(
            src_hbm.at[pl.ds(cbase + t * _TILE, _TILE)], buf, sem)

    def _dcopy(t, buf, sem):
        return pltpu.make_async_copy(
            dst_hbm.at[pl.ds(cbase + t * _TILE, _TILE)], buf, sem)

    def _drain_one(half):
        coff = half * _LANES
        bvec = bvec1 if half else bvec0
        ch = cur[pl.ds(coff, _LANES)]
        full = (ch - bvec) >= _BATCH

        @pl.when(plsc.all_reduce_population_count(full)[0] > 0)
        def _():
            bl = plsc.all_reduce_ffs(full)[0]
            bg = bl + half * _LANES
            dch = dc[pl.ds(coff, _LANES)]
            dcb = jnp.sum(jnp.where(iota == bl, dch, 0))
            off = (wid * _NW + bg) * _REGCAP + dcb * _BATCH
            pltpu.sync_copy(stg_s.at[pl.ds(bg * _CAP, _BATCH)],
                            bsrc_hbm.at[pl.ds(off, _BATCH)])
            pltpu.sync_copy(stg_d.at[pl.ds(bg * _CAP, _BATCH)],
                            bdst_hbm.at[pl.ds(off, _BATCH)])
            for k in range((_CAP - _BATCH) // _LANES):
                stg_s[pl.ds(bg * _CAP + k * _LANES, _LANES)] = \
                    stg_s[pl.ds(bg * _CAP + _BATCH + k * _LANES, _LANES)]
                stg_d[pl.ds(bg * _CAP + k * _LANES, _LANES)] = \
                    stg_d[pl.ds(bg * _CAP + _BATCH + k * _LANES, _LANES)]
            cur[pl.ds(coff, _LANES)] = ch - \
                jnp.where(iota == bl, _BATCH, 0)
            dc[pl.ds(coff, _LANES)] = dch + jnp.where(iota == bl, 1, 0)

    def _scan_tile(sbuf, dbuf):
        def _v(v, c):
            sv = sbuf[pl.ds(v * _LANES, _LANES)]
            dv = dbuf[pl.ds(v * _LANES, _LANES)]
            b16 = ((sv.astype(jnp.float32) + 0.5) *
                   (1.0 / _RANGE)).astype(jnp.int32)
            sk, perm = plsc.sort_key_val(b16, iota)
            svs = _vgather(sv, perm)
            dvs = _vgather(dv, perm)
            prev = _vgather(sk, jnp.maximum(iota - 1, 0))
            first = (iota == 0) | (sk != prev)
            rank = iota - plsc.cummax(jnp.where(first, iota, 0))
            base = plsc.load_gather(cur, [sk])
            pos = base + rank
            plsc.store_scatter(stg_s, [pos], svs)
            plsc.store_scatter(stg_d, [pos], dvs)
            nxt = _vgather(sk, jnp.minimum(iota + 1, _LANES - 1))
            last = (iota == _LANES - 1) | (sk != nxt)
            plsc.store_scatter(cur, [sk], pos + 1, mask=last)
            _drain_one(0)
            _drain_one(1)
            return c
        lax.fori_loop(0, _TILE // _LANES, _v, 0)

    _scopy(0, sbuf0, sem_s0).start()
    _dcopy(0, dbuf0, sem_d0).start()
    _scopy(1, sbuf1, sem_s1).start()
    _dcopy(1, dbuf1, sem_d1).start()

    def _pair(i, c):
        t0 = i * 2
        for b in range(2):
            t = t0 + b
            sbuf = sbuf0 if b == 0 else sbuf1
            dbuf = dbuf0 if b == 0 else dbuf1
            sem_s = sem_s0 if b == 0 else sem_s1
            sem_d = sem_d0 if b == 0 else sem_d1
            _scopy(t, sbuf, sem_s).wait()
            _dcopy(t, dbuf, sem_d).wait()
            _scan_tile(sbuf, dbuf)

            @pl.when(t + 2 < _NT2)
            def _(t=t, sbuf=sbuf, dbuf=dbuf, sem_s=sem_s, sem_d=sem_d):
                _scopy(t + 2, sbuf, sem_s).start()
                _dcopy(t + 2, dbuf, sem_d).start()
        return c
    lax.fori_loop(0, _NT2 // 2, _pair, 0)

    cur0 = cur[pl.ds(0, _LANES)]
    cur1 = cur[pl.ds(_LANES, _LANES)]
    dc0 = dc[pl.ds(0, _LANES)]
    dc1 = dc[pl.ds(_LANES, _LANES)]
    cstage[pl.ds(0, _LANES)] = dc0 * _BATCH + cur0 - bvec0
    cstage[pl.ds(_LANES, _LANES)] = dc1 * _BATCH + cur1 - bvec1
    pltpu.sync_copy(cstage, cnts_hbm.at[pl.ds(wid * _NW, _NW)])

    for bg in range(_NW):
        half = bg // _LANES
        bl = bg % _LANES
        dch = dc[pl.ds(half * _LANES, _LANES)]
        dcb = jnp.sum(jnp.where(iota == bl, dch, 0))
        off = (wid * _NW + bg) * _REGCAP + dcb * _BATCH
        pltpu.sync_copy(stg_s.at[pl.ds(bg * _CAP, _CAP)],
                        bsrc_hbm.at[pl.ds(off, _CAP)])
        pltpu.sync_copy(stg_d.at[pl.ds(bg * _CAP, _CAP)],
                        bdst_hbm.at[pl.ds(off, _CAP)])


def _partition(src, dst):
    mesh = plsc.VectorSubcoreMesh(core_axis_name="c", subcore_axis_name="s")
    f = pl.kernel(
        _part_body,
        mesh=mesh,
        compiler_params=_SC_PARAMS,
        out_type=[
            jax.ShapeDtypeStruct((_NW * _NW * _REGCAP,), jnp.int32),
            jax.ShapeDtypeStruct((_NW * _NW * _REGCAP,), jnp.int32),
            jax.ShapeDtypeStruct((_NW * _NW,), jnp.int32),
        ],
        scratch_types=[
            pltpu.VMEM((_TILE,), jnp.int32),
            pltpu.VMEM((_TILE,), jnp.int32),
            pltpu.VMEM((_TILE,), jnp.int32),
            pltpu.VMEM((_TILE,), jnp.int32),
            pltpu.VMEM((_NW * _CAP,), jnp.int32),
            pltpu.VMEM((_NW * _CAP,), jnp.int32),
            pltpu.VMEM((_NW,), jnp.int32),
            pltpu.VMEM((_NW,), jnp.int32),
            pltpu.VMEM((_NW,), jnp.int32),
            pltpu.SemaphoreType.DMA,
            pltpu.SemaphoreType.DMA,
            pltpu.SemaphoreType.DMA,
            pltpu.SemaphoreType.DMA,
        ],
    )
    return f(src, dst)


def _bucket_body(bsrc_hbm, bdst_hbm, cnts_hbm, a_hbm, p_hbm,
                 pooled_hbm, part_hbm,
                 cnts_v,
                 idx_s0, idx_d0, ba0, bp0,
                 idx_s1, idx_d1, ba1, bp1,
                 acc, sums, sem_c,
                 sem_a0, sem_p0, sem_a1, sem_p1,
                 sem_i0, sem_i1):
    cid = lax.axis_index("c")
    sid = lax.axis_index("s")
    wid = sid * 2 + cid
    lo = wid * _RANGE
    iota = lax.iota(jnp.int32, _LANES)
    sets = ((idx_s0, idx_d0, ba0, bp0, sem_a0, sem_p0, sem_i0),
            (idx_s1, idx_d1, ba1, bp1, sem_a1, sem_p1, sem_i1))

    neg = jnp.full((_LANES,), -jnp.inf, jnp.float32)
    zero = jnp.zeros((_LANES,), jnp.float32)

    def _init(i, c):
        acc[pl.ds(i * _LANES, _LANES)] = neg
        return c
    lax.fori_loop(0, _RANGE, _init, 0)
    sums[pl.ds(0, _LANES)] = zero
    sums[pl.ds(_LANES, _LANES)] = zero

    cp = pltpu.make_async_copy(cnts_hbm, cnts_v, sem_c)
    cp.start()
    cp.wait()

    def _lane_bcast(vec, e_rel):
        return _vgather(vec, jnp.full((_LANES,), e_rel, jnp.int32))

    def _stage_start(w2, j, limit, st):
        idx_s, idx_d, ba, bp, sem_a, sem_p, sem_i = st
        base = (w2 * _NW + wid) * _REGCAP + j * _BATCH
        ci = pltpu.make_async_copy(
            bsrc_hbm.at[pl.ds(base, _BATCH)], idx_s, sem_i)
        ci.start()
        cj = pltpu.make_async_copy(
            bdst_hbm.at[pl.ds(base, _BATCH)], idx_d, sem_i)
        cj.start()
        ci.wait()
        cj.wait()
        rel = limit - j * _BATCH

        @pl.when(rel < _BATCH)
        def _():
            for v in range(_BATCH // _LANES):
                mlane = (iota + v * _LANES) < rel
                sv = idx_s[pl.ds(v * _LANES, _LANES)]
                dv = idx_d[pl.ds(v * _LANES, _LANES)]
                idx_s[pl.ds(v * _LANES, _LANES)] = jnp.where(mlane, sv, lo)
                idx_d[pl.ds(v * _LANES, _LANES)] = jnp.where(mlane, dv, 0)
        pltpu.make_async_copy(a_hbm.at[idx_d], ba, sem_a).start()
        pltpu.make_async_copy(p_hbm.at[idx_s], bp, sem_p).start()

    def _wait(st):
        idx_s, idx_d, ba, bp, sem_a, sem_p, _ = st
        pltpu.make_async_copy(a_hbm.at[idx_d], ba, sem_a).wait()
        pltpu.make_async_copy(p_hbm.at[idx_s], bp, sem_p).wait()

    def _edge_update(e_rel, loc16, off, ba, bp):
        sb = _lane_bcast(loc16, e_rel)
        a_row = ba[off + e_rel, :]
        p_row = bp[off + e_rel, :]
        msg = a_row - p_row
        sums[pl.ds(0, _LANES)] = sums[pl.ds(0, _LANES)] + msg
        sums[pl.ds(_LANES, _LANES)] = sums[pl.ds(_LANES, _LANES)] + msg * msg
        ridx = sb * _LANES + iota
        old = plsc.load_gather(acc, [ridx])
        plsc.store_scatter(acc, [ridx], jnp.maximum(old, msg))

    def _process(j, limit, st):
        idx_s, idx_d, ba, bp, _, _, _ = st
        rel = limit - j * _BATCH

        def _vblock(v, c):
            loc16 = idx_s[pl.ds(v * _LANES, _LANES)] - lo

            @pl.when(v * _LANES < rel)
            def _(v=v, loc16=loc16):
                @pl.when(rel >= (v + 1) * _LANES)
                def _():
                    for k in range(_LANES):
                        _edge_update(k, loc16, v * _LANES, ba, bp)

                @pl.when(rel < (v + 1) * _LANES)
                def _():
                    for k in range(_LANES):
                        @pl.when(v * _LANES + k < rel)
                        def _(k=k):
                            _edge_update(k, loc16, v * _LANES, ba, bp)
            return c
        lax.fori_loop(0, _BATCH // _LANES, _vblock, 0)

    def _seg(w2, c):
        cnt = plsc.load_gather(cnts_v, [jnp.full((_LANES,),
                                                 w2 * _NW + wid,
                                                 jnp.int32)])[0]
        nb = (cnt + _BATCH - 1) // _BATCH

        @pl.when(nb > 0)
        def _():
            _stage_start(w2, 0, cnt, sets[0])

            @pl.when(nb > 1)
            def _():
                _stage_start(w2, 1, cnt, sets[1])

            def _pairb(k, c2):
                for b2 in range(2):
                    j0 = k * 2 + b2

                    @pl.when(j0 < nb)
                    def _(j0=j0, b2=b2):
                        _wait(sets[b2])
                        _process(j0, cnt, sets[b2])

                        @pl.when(j0 + 2 < nb)
                        def _(j0=j0, b2=b2):
                            _stage_start(w2, j0 + 2, cnt, sets[b2])
                return c2
            lax.fori_loop(0, (nb + 1) // 2, _pairb, 0)
        return c
    lax.fori_loop(0, _NW, _seg, 0)

    pltpu.sync_copy(acc, pooled_hbm.at[pl.ds(lo * _LANES, _RANGE * _LANES)])
    pltpu.sync_copy(sums, part_hbm.at[pl.ds(wid * 2 * _LANES, 2 * _LANES)])


def _bucket_pass(bsrc, bdst, cnts, a_tab, p_tab):
    mesh = plsc.VectorSubcoreMesh(core_axis_name="c", subcore_axis_name="s")
    f = pl.kernel(
        _bucket_body,
        mesh=mesh,
        compiler_params=_SC_PARAMS,
        out_type=[
            jax.ShapeDtypeStruct((_N * _LANES,), jnp.float32),
            jax.ShapeDtypeStruct((_NW * 2 * _LANES,), jnp.float32),
        ],
        scratch_types=[
            pltpu.VMEM((_NW * _NW,), jnp.int32),
            pltpu.VMEM((_BATCH,), jnp.int32),
            pltpu.VMEM((_BATCH,), jnp.int32),
            pltpu.VMEM((_BATCH, _LANES), jnp.float32),
            pltpu.VMEM((_BATCH, _LANES), jnp.float32),
            pltpu.VMEM((_BATCH,), jnp.int32),
            pltpu.VMEM((_BATCH,), jnp.int32),
            pltpu.VMEM((_BATCH, _LANES), jnp.float32),
            pltpu.VMEM((_BATCH, _LANES), jnp.float32),
            pltpu.VMEM((_RANGE * _LANES,), jnp.float32),
            pltpu.VMEM((2 * _LANES,), jnp.float32),
            pltpu.SemaphoreType.DMA,
            pltpu.SemaphoreType.DMA,
            pltpu.SemaphoreType.DMA,
            pltpu.SemaphoreType.DMA,
            pltpu.SemaphoreType.DMA,
            pltpu.SemaphoreType.DMA,
            pltpu.SemaphoreType.DMA,
        ],
    )
    return f(bsrc, bdst, cnts, a_tab, p_tab)


def kernel(node, features, edges, W, gamma, beta):
    n = node.shape[0]
    e = edges.shape[0]
    src = edges[:, 0]
    dst = edges[:, 1]

    u = jnp.concatenate(
        [features, node, jnp.zeros((n, 5), jnp.float32)], axis=1)
    wbig = jnp.zeros((24, 32), jnp.float32)
    wbig = wbig.at[:19, :16].set(W.T)
    wbig = wbig.at[16:19, 16:].set(W[:, 16:19].T)

    a_tab, p_tab = _build_tables(u, wbig)
    bsrc, bdst, cnts = _partition(src, dst)
    pooled_flat, partials = _bucket_pass(bsrc, bdst, cnts, a_tab, p_tab)
    raw = pooled_flat.reshape(n, 16)
    part = partials.reshape(_NW, 2, _LANES)
    out = _finalize(raw, part[:, 0, :], part[:, 1, :],
                    gamma.reshape(1, 16), beta.reshape(1, 16), float(e))

    present = raw[:, 0] > -jnp.inf

    def _compact(o):
        idxs = jnp.nonzero(present, size=n, fill_value=0)[0]
        cnt = jnp.sum(present.astype(jnp.int32))
        o2 = jnp.take(o, idxs, axis=0)
        return jnp.where((jnp.arange(n) < cnt)[:, None], o2, -jnp.inf)

    return lax.cond(jnp.all(present), lambda o: o, _compact, out)


# speedup vs baseline: 4.1428x; 1.2098x over previous
import functools

import jax
import jax.numpy as jnp
from jax import lax
from jax.experimental import pallas as pl
from jax.experimental.pallas import tpu as pltpu
from jax.experimental.pallas import tpu_sc as plsc

_N = 100000
_E = 3200000
_NW = 32
_RANGE = _N // _NW
_CHUNK = _E // _NW
_TILE = 2000
_NT2 = _CHUNK // _TILE
_BATCH = 256
_CAP = 544
_REGCAP = _CHUNK + 576
_LANES = 16

_SC_PARAMS = pltpu.CompilerParams(
    needs_layout_passes=False, use_tc_tiling_on_sc=False)
_GD = lax.GatherDimensionNumbers(
    offset_dims=(), collapsed_slice_dims=(0,), start_index_map=(0,))


def _vgather(vec, idx):
    return lax.gather(vec, idx.reshape(_LANES, 1), _GD, (1,),
                      mode=lax.GatherScatterMode.PROMISE_IN_BOUNDS)


def _tables_body(u_ref, w_ref, a_ref, p_ref):
    ap = jnp.dot(u_ref[...], w_ref[...], preferred_element_type=jnp.float32)
    a_ref[...] = ap[:, :16]
    p_ref[...] = ap[:, 16:]


def _build_tables(u, wbig):
    n = u.shape[0]
    blk = 1000
    return pl.pallas_call(
        _tables_body,
        grid=(n // blk,),
        in_specs=[
            pl.BlockSpec((blk, 24), lambda i: (i, 0)),
            pl.BlockSpec((24, 32), lambda i: (0, 0)),
        ],
        out_specs=[
            pl.BlockSpec((blk, 16), lambda i: (i, 0)),
            pl.BlockSpec((blk, 16), lambda i: (i, 0)),
        ],
        out_shape=[
            jax.ShapeDtypeStruct((n, 16), jnp.float32),
            jax.ShapeDtypeStruct((n, 16), jnp.float32),
        ],
    )(u, wbig)


def _final_body(e_f, raw_ref, sums_ref, sq_ref, g_ref, b_ref, o_ref):
    s = jnp.sum(sums_ref[...], axis=0, keepdims=True)
    q = jnp.sum(sq_ref[...], axis=0, keepdims=True)
    mean = s / e_f
    var = q / e_f - mean * mean
    rstd = lax.rsqrt(var + 1e-5)
    raw = raw_ref[...]
    aff = (raw - mean) * (rstd * g_ref[...]) + b_ref[...]
    o_ref[...] = jnp.where(raw == -jnp.inf, raw, aff)


def _finalize(raw, sums, sumsqs, gamma, beta, e_f):
    n = raw.shape[0]
    blk = 1000
    return pl.pallas_call(
        functools.partial(_final_body, e_f),
        grid=(n // blk,),
        in_specs=[
            pl.BlockSpec((blk, 16), lambda i: (i, 0)),
            pl.BlockSpec((_NW, 16), lambda i: (0, 0)),
            pl.BlockSpec((_NW, 16), lambda i: (0, 0)),
            pl.BlockSpec((1, 16), lambda i: (0, 0)),
            pl.BlockSpec((1, 16), lambda i: (0, 0)),
        ],
        out_specs=pl.BlockSpec((blk, 16), lambda i: (i, 0)),
        out_shape=jax.ShapeDtypeStruct((n, 16), jnp.float32),
    )(raw, sums, sumsqs, gamma, beta)


def _part_body(src_hbm, dst_hbm,
               bsrc_hbm, bdst_hbm, cnts_hbm,
               sbuf0, sbuf1, dbuf0, dbuf1,
               stg_s, stg_d, cur, dc, cstage,
               sem_s0, sem_s1, sem_d0, sem_d1):
    cid = lax.axis_index("c")
    sid = lax.axis_index("s")
    wid = sid * 2 + cid
    iota = lax.iota(jnp.int32, _LANES)
    bvec0 = iota * _CAP
    bvec1 = (iota + _LANES) * _CAP

    cur[pl.ds(0, _LANES)] = bvec0
    cur[pl.ds(_LANES, _LANES)] = bvec1
    dc[pl.ds(0, _LANES)] = jnp.zeros((_LANES,), jnp.int32)
    dc[pl.ds(_LANES, _LANES)] = jnp.zeros((_LANES,), jnp.int32)

    cbase = wid * _CHUNK

    def _scopy(t, buf, sem):
        return pltpu.make_async_copy(
            src_hbm.at[pl.ds(cbase + t * _TILE, _TILE)], buf, sem)

    def _dcopy(t, buf, sem):
        return pltpu.make_async_copy(
            dst_hbm.at[pl.ds(cbase + t * _TILE, _TILE)], buf, sem)

    def _drain_one(half):
        coff = half * _LANES
        bvec = bvec1 if half else bvec0
        ch = cur[pl.ds(coff, _LANES)]
        full = (ch - bvec) >= _BATCH

        @pl.when(plsc.all_reduce_population_count(full)[0] > 0)
        def _():
            bl = plsc.all_reduce_ffs(full)[0]
            bg = bl + half * _LANES
            dch = dc[pl.ds(coff, _LANES)]
            dcb = jnp.sum(jnp.where(iota == bl, dch, 0))
            off = (wid * _NW + bg) * _REGCAP + dcb * _BATCH
            pltpu.sync_copy(stg_s.at[pl.ds(bg * _CAP, _BATCH)],
                            bsrc_hbm.at[pl.ds(off, _BATCH)])
            pltpu.sync_copy(stg_d.at[pl.ds(bg * _CAP, _BATCH)],
                            bdst_hbm.at[pl.ds(off, _BATCH)])
            for k in range((_CAP - _BATCH) // _LANES):
                stg_s[pl.ds(bg * _CAP + k * _LANES, _LANES)] = \
                    stg_s[pl.ds(bg * _CAP + _BATCH + k * _LANES, _LANES)]
                stg_d[pl.ds(bg * _CAP + k * _LANES, _LANES)] = \
                    stg_d[pl.ds(bg * _CAP + _BATCH + k * _LANES, _LANES)]
            cur[pl.ds(coff, _LANES)] = ch - \
                jnp.where(iota == bl, _BATCH, 0)
            dc[pl.ds(coff, _LANES)] = dch + jnp.where(iota == bl, 1, 0)

    def _scan_tile(sbuf, dbuf):
        def _v(v, c):
            sv = sbuf[pl.ds(v * _LANES, _LANES)]
            dv = dbuf[pl.ds(v * _LANES, _LANES)]
            b16 = ((sv.astype(jnp.float32) + 0.5) *
                   (1.0 / _RANGE)).astype(jnp.int32)
            sk, perm = plsc.sort_key_val(b16, iota)
            svs = _vgather(sv, perm)
            dvs = _vgather(dv, perm)
            prev = _vgather(sk, jnp.maximum(iota - 1, 0))
            first = (iota == 0) | (sk != prev)
            rank = iota - plsc.cummax(jnp.where(first, iota, 0))
            base = plsc.load_gather(cur, [sk])
            pos = base + rank
            plsc.store_scatter(stg_s, [pos], svs)
            plsc.store_scatter(stg_d, [pos], dvs)
            nxt = _vgather(sk, jnp.minimum(iota + 1, _LANES - 1))
            last = (iota == _LANES - 1) | (sk != nxt)
            plsc.store_scatter(cur, [sk], pos + 1, mask=last)
            _drain_one(0)
            _drain_one(1)
            return c
        lax.fori_loop(0, _TILE // _LANES, _v, 0)

    _scopy(0, sbuf0, sem_s0).start()
    _dcopy(0, dbuf0, sem_d0).start()
    _scopy(1, sbuf1, sem_s1).start()
    _dcopy(1, dbuf1, sem_d1).start()

    def _pair(i, c):
        t0 = i * 2
        for b in range(2):
            t = t0 + b
            sbuf = sbuf0 if b == 0 else sbuf1
            dbuf = dbuf0 if b == 0 else dbuf1
            sem_s = sem_s0 if b == 0 else sem_s1
            sem_d = sem_d0 if b == 0 else sem_d1
            _scopy(t, sbuf, sem_s).wait()
            _dcopy(t, dbuf, sem_d).wait()
            _scan_tile(sbuf, dbuf)

            @pl.when(t + 2 < _NT2)
            def _(t=t, sbuf=sbuf, dbuf=dbuf, sem_s=sem_s, sem_d=sem_d):
                _scopy(t + 2, sbuf, sem_s).start()
                _dcopy(t + 2, dbuf, sem_d).start()
        return c
    lax.fori_loop(0, _NT2 // 2, _pair, 0)

    cur0 = cur[pl.ds(0, _LANES)]
    cur1 = cur[pl.ds(_LANES, _LANES)]
    dc0 = dc[pl.ds(0, _LANES)]
    dc1 = dc[pl.ds(_LANES, _LANES)]
    cstage[pl.ds(0, _LANES)] = dc0 * _BATCH + cur0 - bvec0
    cstage[pl.ds(_LANES, _LANES)] = dc1 * _BATCH + cur1 - bvec1
    pltpu.sync_copy(cstage, cnts_hbm.at[pl.ds(wid * _NW, _NW)])

    for bg in range(_NW):
        half = bg // _LANES
        bl = bg % _LANES
        dch = dc[pl.ds(half * _LANES, _LANES)]
        dcb = jnp.sum(jnp.where(iota == bl, dch, 0))
        off = (wid * _NW + bg) * _REGCAP + dcb * _BATCH
        pltpu.sync_copy(stg_s.at[pl.ds(bg * _CAP, _CAP)],
                        bsrc_hbm.at[pl.ds(off, _CAP)])
        pltpu.sync_copy(stg_d.at[pl.ds(bg * _CAP, _CAP)],
                        bdst_hbm.at[pl.ds(off, _CAP)])


def _partition(src, dst):
    mesh = plsc.VectorSubcoreMesh(core_axis_name="c", subcore_axis_name="s")
    f = pl.kernel(
        _part_body,
        mesh=mesh,
        compiler_params=_SC_PARAMS,
        out_type=[
            jax.ShapeDtypeStruct((_NW * _NW * _REGCAP,), jnp.int32),
            jax.ShapeDtypeStruct((_NW * _NW * _REGCAP,), jnp.int32),
            jax.ShapeDtypeStruct((_NW * _NW,), jnp.int32),
        ],
        scratch_types=[
            pltpu.VMEM((_TILE,), jnp.int32),
            pltpu.VMEM((_TILE,), jnp.int32),
            pltpu.VMEM((_TILE,), jnp.int32),
            pltpu.VMEM((_TILE,), jnp.int32),
            pltpu.VMEM((_NW * _CAP,), jnp.int32),
            pltpu.VMEM((_NW * _CAP,), jnp.int32),
            pltpu.VMEM((_NW,), jnp.int32),
            pltpu.VMEM((_NW,), jnp.int32),
            pltpu.VMEM((_NW,), jnp.int32),
            pltpu.SemaphoreType.DMA,
            pltpu.SemaphoreType.DMA,
            pltpu.SemaphoreType.DMA,
            pltpu.SemaphoreType.DMA,
        ],
    )
    return f(src, dst)


def _bucket_body(bsrc_hbm, bdst_hbm, cnts_hbm, a_hbm, p_hbm,
                 pooled_hbm, part_hbm,
                 cnts_v,
                 idx_s0, idx_d0, ba0, bp0,
                 idx_s1, idx_d1, ba1, bp1,
                 acc, acc2, sums, sem_c,
                 sem_a0, sem_p0, sem_a1, sem_p1,
                 sem_i0, sem_i1):
    cid = lax.axis_index("c")
    sid = lax.axis_index("s")
    wid = sid * 2 + cid
    lo = wid * _RANGE
    iota = lax.iota(jnp.int32, _LANES)
    sets = ((idx_s0, idx_d0, ba0, bp0, sem_a0, sem_p0, sem_i0),
            (idx_s1, idx_d1, ba1, bp1, sem_a1, sem_p1, sem_i1))

    neg = jnp.full((_LANES,), -jnp.inf, jnp.float32)
    zero = jnp.zeros((_LANES,), jnp.float32)

    def _init(i, c):
        acc[pl.ds(i * _LANES, _LANES)] = neg
        acc2[pl.ds(i * _LANES, _LANES)] = neg
        return c
    lax.fori_loop(0, _RANGE, _init, 0)
    sums[pl.ds(0, _LANES)] = zero
    sums[pl.ds(_LANES, _LANES)] = zero

    cp = pltpu.make_async_copy(cnts_hbm, cnts_v, sem_c)
    cp.start()
    cp.wait()

    def _lane_bcast(vec, e_rel):
        return _vgather(vec, jnp.full((_LANES,), e_rel, jnp.int32))

    def _stage_start(w2, j, limit, st):
        idx_s, idx_d, ba, bp, sem_a, sem_p, sem_i = st
        base = (w2 * _NW + wid) * _REGCAP + j * _BATCH
        ci = pltpu.make_async_copy(
            bsrc_hbm.at[pl.ds(base, _BATCH)], idx_s, sem_i)
        ci.start()
        cj = pltpu.make_async_copy(
            bdst_hbm.at[pl.ds(base, _BATCH)], idx_d, sem_i)
        cj.start()
        ci.wait()
        cj.wait()
        rel = limit - j * _BATCH

        @pl.when(rel < _BATCH)
        def _():
            for v in range(_BATCH // _LANES):
                mlane = (iota + v * _LANES) < rel
                sv = idx_s[pl.ds(v * _LANES, _LANES)]
                dv = idx_d[pl.ds(v * _LANES, _LANES)]
                idx_s[pl.ds(v * _LANES, _LANES)] = jnp.where(mlane, sv, lo)
                idx_d[pl.ds(v * _LANES, _LANES)] = jnp.where(mlane, dv, 0)
        pltpu.make_async_copy(a_hbm.at[idx_d], ba, sem_a).start()
        pltpu.make_async_copy(p_hbm.at[idx_s], bp, sem_p).start()

    def _wait(st):
        idx_s, idx_d, ba, bp, sem_a, sem_p, _ = st
        pltpu.make_async_copy(a_hbm.at[idx_d], ba, sem_a).wait()
        pltpu.make_async_copy(p_hbm.at[idx_s], bp, sem_p).wait()

    def _edge_update(e_rel, loc16, off, ba, bp, accx, s, q, rel):
        sb = _lane_bcast(loc16, e_rel)
        a_row = ba[off + e_rel, :]
        p_row = bp[off + e_rel, :]
        msg = a_row - p_row
        ridx = sb * _LANES + iota
        if rel is None:
            s = s + msg
            q = q + msg * msg
            old = plsc.load_gather(accx, [ridx])
            plsc.store_scatter(accx, [ridx], jnp.maximum(old, msg))
        else:
            ok = jnp.full((_LANES,), e_rel, jnp.int32) < rel
            msk = jnp.where(ok, msg, 0.0)
            s = s + msk
            q = q + msk * msk
            old = plsc.load_gather(accx, [ridx])
            plsc.store_scatter(accx, [ridx], jnp.maximum(old, msg), mask=ok)
        return s, q

    def _process(j, limit, st, masked):
        idx_s, idx_d, ba, bp, _, _, _ = st
        rel = limit - j * _BATCH

        def _vblock(v, carry):
            s, q = carry
            loc16 = idx_s[pl.ds(v * _LANES, _LANES)] - lo
            relv = jnp.full((_LANES,), rel - v * _LANES,
                            jnp.int32) if masked else None
            for k in range(_LANES):
                accx = acc if (k % 2 == 0) else acc2
                s, q = _edge_update(k, loc16, v * _LANES, ba, bp,
                                    accx, s, q, relv)
            return s, q
        zz = jnp.zeros((_LANES,), jnp.float32)
        s, q = lax.fori_loop(0, _BATCH // _LANES, _vblock, (zz, zz))
        sums[pl.ds(0, _LANES)] = sums[pl.ds(0, _LANES)] + s
        sums[pl.ds(_LANES, _LANES)] = sums[pl.ds(_LANES, _LANES)] + q

    def _seg(w2, c):
        cnt = plsc.load_gather(cnts_v, [jnp.full((_LANES,),
                                                 w2 * _NW + wid,
                                                 jnp.int32)])[0]
        nb = (cnt + _BATCH - 1) // _BATCH
        nbf = cnt // _BATCH

        @pl.when(nb > 0)
        def _():
            _stage_start(w2, 0, cnt, sets[0])

            @pl.when(nb > 1)
            def _():
                _stage_start(w2, 1, cnt, sets[1])

            def _pairb(k, c2):
                for b2 in range(2):
                    j0 = k * 2 + b2

                    @pl.when(j0 < nb)
                    def _(j0=j0, b2=b2):
                        _wait(sets[b2])

                        @pl.when(j0 < nbf)
                        def _(j0=j0, b2=b2):
                            _process(j0, cnt, sets[b2], False)

                        @pl.when(j0 >= nbf)
                        def _(j0=j0, b2=b2):
                            _process(j0, cnt, sets[b2], True)

                        @pl.when(j0 + 2 < nb)
                        def _(j0=j0, b2=b2):
                            _stage_start(w2, j0 + 2, cnt, sets[b2])
                return c2
            lax.fori_loop(0, (nb + 1) // 2, _pairb, 0)
        return c
    lax.fori_loop(0, _NW, _seg, 0)

    def _merge(i, c):
        acc[pl.ds(i * _LANES, _LANES)] = jnp.maximum(
            acc[pl.ds(i * _LANES, _LANES)], acc2[pl.ds(i * _LANES, _LANES)])
        return c
    lax.fori_loop(0, _RANGE, _merge, 0)
    pltpu.sync_copy(acc, pooled_hbm.at[pl.ds(lo * _LANES, _RANGE * _LANES)])
    pltpu.sync_copy(sums, part_hbm.at[pl.ds(wid * 2 * _LANES, 2 * _LANES)])


def _bucket_pass(bsrc, bdst, cnts, a_tab, p_tab):
    mesh = plsc.VectorSubcoreMesh(core_axis_name="c", subcore_axis_name="s")
    f = pl.kernel(
        _bucket_body,
        mesh=mesh,
        compiler_params=_SC_PARAMS,
        out_type=[
            jax.ShapeDtypeStruct((_N * _LANES,), jnp.float32),
            jax.ShapeDtypeStruct((_NW * 2 * _LANES,), jnp.float32),
        ],
        scratch_types=[
            pltpu.VMEM((_NW * _NW,), jnp.int32),
            pltpu.VMEM((_BATCH,), jnp.int32),
            pltpu.VMEM((_BATCH,), jnp.int32),
            pltpu.VMEM((_BATCH, _LANES), jnp.float32),
            pltpu.VMEM((_BATCH, _LANES), jnp.float32),
            pltpu.VMEM((_BATCH,), jnp.int32),
            pltpu.VMEM((_BATCH,), jnp.int32),
            pltpu.VMEM((_BATCH, _LANES), jnp.float32),
            pltpu.VMEM((_BATCH, _LANES), jnp.float32),
            pltpu.VMEM((_RANGE * _LANES,), jnp.float32),
            pltpu.VMEM((_RANGE * _LANES,), jnp.float32),
            pltpu.VMEM((2 * _LANES,), jnp.float32),
            pltpu.SemaphoreType.DMA,
            pltpu.SemaphoreType.DMA,
            pltpu.SemaphoreType.DMA,
            pltpu.SemaphoreType.DMA,
            pltpu.SemaphoreType.DMA,
            pltpu.SemaphoreType.DMA,
            pltpu.SemaphoreType.DMA,
        ],
    )
    return f(bsrc, bdst, cnts, a_tab, p_tab)


def kernel(node, features, edges, W, gamma, beta):
    n = node.shape[0]
    e = edges.shape[0]
    src = edges[:, 0]
    dst = edges[:, 1]

    u = jnp.concatenate(
        [features, node, jnp.zeros((n, 5), jnp.float32)], axis=1)
    wbig = jnp.zeros((24, 32), jnp.float32)
    wbig = wbig.at[:19, :16].set(W.T)
    wbig = wbig.at[16:19, 16:].set(W[:, 16:19].T)

    a_tab, p_tab = _build_tables(u, wbig)
    bsrc, bdst, cnts = _partition(src, dst)
    pooled_flat, partials = _bucket_pass(bsrc, bdst, cnts, a_tab, p_tab)
    raw = pooled_flat.reshape(n, 16)
    part = partials.reshape(_NW, 2, _LANES)
    out = _finalize(raw, part[:, 0, :], part[:, 1, :],
                    gamma.reshape(1, 16), beta.reshape(1, 16), float(e))

    present = raw[:, 0] > -jnp.inf

    def _compact(o):
        idxs = jnp.nonzero(present, size=n, fill_value=0)[0]
        cnt = jnp.sum(present.astype(jnp.int32))
        o2 = jnp.take(o, idxs, axis=0)
        return jnp.where((jnp.arange(n) < cnt)[:, None], o2, -jnp.inf)

    return lax.cond(jnp.all(present), lambda o: o, _compact, out)
